# baseline (speedup 1.0000x reference)
import functools

import jax
import jax.numpy as jnp
import numpy as np
from jax import lax
from jax.experimental import pallas as pl
from jax.experimental.pallas import tpu as pltpu
from jax.experimental.pallas import tpu_sc as plsc


@functools.lru_cache(maxsize=None)
def _make_sc_gather(B, D):
    info = plsc.get_sparse_core_info()
    NC, NS = info.num_cores, info.num_subcores
    NW = NC * NS
    assert B % (8 * NW) == 0
    b_per_w = B // NW
    mesh = plsc.VectorSubcoreMesh(core_axis_name="c", subcore_axis_name="s")

    @functools.partial(
        pl.kernel,
        mesh=mesh,
        compiler_params=pltpu.CompilerParams(use_tc_tiling_on_sc=False),
        out_type=[
            jax.ShapeDtypeStruct((B, D), jnp.float32),
            jax.ShapeDtypeStruct((B, D), jnp.float32),
        ],
        scratch_types=[
            pltpu.VMEM((b_per_w,), jnp.int32),
            pltpu.VMEM((b_per_w,), jnp.int32),
            pltpu.VMEM((b_per_w, D), jnp.float32),
            pltpu.VMEM((b_per_w, D), jnp.float32),
            pltpu.SemaphoreType.DMA,
            pltpu.SemaphoreType.DMA,
        ],
    )
    def sc_gather(xi_hbm, yi_hbm, xt_hbm, yt_hbm, xout_hbm, yout_hbm,
                  xi_v, yi_v, xrows_v, yrows_v, semx, semy):
        wid = lax.axis_index("s") * NC + lax.axis_index("c")
        base = wid * b_per_w
        pltpu.sync_copy(xi_hbm.at[pl.ds(base, b_per_w)], xi_v)
        pltpu.sync_copy(yi_hbm.at[pl.ds(base, b_per_w)], yi_v)
        cx = pltpu.async_copy(xt_hbm.at[xi_v], xrows_v, semx)
        cy = pltpu.async_copy(yt_hbm.at[yi_v], yrows_v, semy)
        cx.wait()
        cy.wait()
        pltpu.sync_copy(xrows_v, xout_hbm.at[pl.ds(base, b_per_w)])
        pltpu.sync_copy(yrows_v, yout_hbm.at[pl.ds(base, b_per_w)])

    return sc_gather


_NB = 4


def _outer_sum_body(nb, x_ref, y_ref, e_ref, f_ref, o_ref):
    e = e_ref[...]
    f = f_ref[...]
    for m in range(nb):
        yp = lax.dot_general(y_ref[m], e, (((0,), (0,)), ((), ())),
                             preferred_element_type=jnp.float32,
                             precision=lax.Precision.HIGHEST)
        xp = lax.dot_general(x_ref[m], f, (((0,), (0,)), ((), ())),
                             preferred_element_type=jnp.float32,
                             precision=lax.Precision.HIGHEST)
        o_ref[m] = yp + xp


@functools.lru_cache(maxsize=None)
def _make_outer_sum(N, S, D, nb):
    SS = S * S
    grid = (N // nb,)
    return pl.pallas_call(
        functools.partial(_outer_sum_body, nb),
        grid=grid,
        in_specs=[
            pl.BlockSpec((nb, S, D), lambda i: (i, 0, 0)),
            pl.BlockSpec((nb, S, D), lambda i: (i, 0, 0)),
            pl.BlockSpec((S, SS), lambda i: (0, 0)),
            pl.BlockSpec((S, SS), lambda i: (0, 0)),
        ],
        out_specs=pl.BlockSpec((nb, D, SS), lambda i: (i, 0, 0)),
        out_shape=jax.ShapeDtypeStruct((N, D, SS), jnp.float32),
    )


def kernel(y_indexes, x_indexes, x_table, y_table):
    N, S = x_indexes.shape
    D = x_table.shape[1]
    B = N * S

    xi = x_indexes.reshape(B).astype(jnp.int32)
    yi = y_indexes.reshape(B).astype(jnp.int32)

    x_emb, y_emb = _make_sc_gather(B, D)(xi, yi, x_table, y_table)

    eye = np.eye(S, dtype=np.float32)
    e_mat = jnp.asarray(np.kron(eye, np.ones((1, S), np.float32)))
    f_mat = jnp.asarray(np.kron(np.ones((1, S), np.float32), eye))

    out = _make_outer_sum(N, S, D, _NB)(
        x_emb.reshape(N, S, D), y_emb.reshape(N, S, D), e_mat, f_mat)
    return out.reshape(N, D, S, S)


# speedup vs baseline: 1.2146x; 1.2146x over previous
import functools

import jax
import jax.numpy as jnp
import numpy as np
from jax import lax
from jax.experimental import pallas as pl
from jax.experimental.pallas import tpu as pltpu
from jax.experimental.pallas import tpu_sc as plsc


@functools.lru_cache(maxsize=None)
def _make_sc_gather(N, S, D):
    B = N * S
    info = plsc.get_sparse_core_info()
    NC, NS = info.num_cores, info.num_subcores
    NW = NC * NS
    assert B % (8 * NW) == 0
    b_per_w = B // NW
    n_per_w = N // NW
    mesh = plsc.VectorSubcoreMesh(core_axis_name="c", subcore_axis_name="s")

    @functools.partial(
        pl.kernel,
        mesh=mesh,
        compiler_params=pltpu.CompilerParams(use_tc_tiling_on_sc=False),
        out_type=jax.ShapeDtypeStruct((N, 2 * S, D), jnp.float32),
        scratch_types=[
            pltpu.VMEM((b_per_w,), jnp.int32),
            pltpu.VMEM((b_per_w,), jnp.int32),
            pltpu.VMEM((b_per_w, D), jnp.float32),
            pltpu.VMEM((b_per_w, D), jnp.float32),
            pltpu.SemaphoreType.DMA,
            pltpu.SemaphoreType.DMA,
        ],
    )
    def sc_gather(yi_hbm, xi_hbm, yt_hbm, xt_hbm, z_hbm,
                  yi_v, xi_v, yrows_v, xrows_v, semg, semo):
        wid = lax.axis_index("s") * NC + lax.axis_index("c")
        base = wid * b_per_w
        pltpu.sync_copy(yi_hbm.at[pl.ds(base, b_per_w)], yi_v)
        pltpu.sync_copy(xi_hbm.at[pl.ds(base, b_per_w)], xi_v)
        cy = pltpu.async_copy(yt_hbm.at[yi_v], yrows_v, semg)
        cx = pltpu.async_copy(xt_hbm.at[xi_v], xrows_v, semg)
        cy.wait()
        cx.wait()
        n0 = wid * n_per_w
        outs = []
        for m in range(n_per_w):
            outs.append(pltpu.async_copy(
                yrows_v.at[pl.ds(m * S, S)], z_hbm.at[n0 + m, pl.ds(0, S)],
                semo))
            outs.append(pltpu.async_copy(
                xrows_v.at[pl.ds(m * S, S)], z_hbm.at[n0 + m, pl.ds(S, S)],
                semo))
        for c in outs:
            c.wait()

    return sc_gather


_NB = 4


def _outer_sum_body(nb, z_ref, g_ref, o_ref):
    g = g_ref[...]
    for m in range(nb):
        o_ref[m] = lax.dot_general(z_ref[m], g, (((0,), (0,)), ((), ())),
                                   preferred_element_type=jnp.float32)


@functools.lru_cache(maxsize=None)
def _make_outer_sum(N, S, D, nb):
    SS = S * S
    return pl.pallas_call(
        functools.partial(_outer_sum_body, nb),
        grid=(N // nb,),
        in_specs=[
            pl.BlockSpec((nb, 2 * S, D), lambda i: (i, 0, 0)),
            pl.BlockSpec((2 * S, SS), lambda i: (0, 0)),
        ],
        out_specs=pl.BlockSpec((nb, D, SS), lambda i: (i, 0, 0)),
        out_shape=jax.ShapeDtypeStruct((N, D, SS), jnp.float32),
    )


@functools.lru_cache(maxsize=None)
def _g_matrix(S):
    eye = np.eye(S, dtype=np.float32)
    e_mat = np.kron(eye, np.ones((1, S), np.float32))
    f_mat = np.kron(np.ones((1, S), np.float32), eye)
    return np.concatenate([e_mat, f_mat], axis=0)


def kernel(y_indexes, x_indexes, x_table, y_table):
    N, S = x_indexes.shape
    D = x_table.shape[1]
    B = N * S

    yi = y_indexes.reshape(B).astype(jnp.int32)
    xi = x_indexes.reshape(B).astype(jnp.int32)

    z = _make_sc_gather(N, S, D)(yi, xi, y_table, x_table)

    g_mat = jnp.asarray(_g_matrix(S))
    out = _make_outer_sum(N, S, D, _NB)(z, g_mat)
    return out.reshape(N, D, S, S)


# speedup vs baseline: 2.7149x; 2.2352x over previous
import functools

import jax
import jax.numpy as jnp
from jax import lax
from jax.experimental import pallas as pl
from jax.experimental.pallas import tpu as pltpu
from jax.experimental.pallas import tpu_sc as plsc


@functools.lru_cache(maxsize=None)
def _make_sc_gather(B, D):
    info = plsc.get_sparse_core_info()
    NC, NS = info.num_cores, info.num_subcores
    NW = NC * NS
    assert B % (8 * NW) == 0
    b_per_w = B // NW
    mesh = plsc.VectorSubcoreMesh(core_axis_name="c", subcore_axis_name="s")

    @functools.partial(
        pl.kernel,
        mesh=mesh,
        compiler_params=pltpu.CompilerParams(use_tc_tiling_on_sc=False),
        out_type=[
            jax.ShapeDtypeStruct((B, D), jnp.float32),
            jax.ShapeDtypeStruct((B, D), jnp.float32),
        ],
        scratch_types=[
            pltpu.VMEM((b_per_w,), jnp.int32),
            pltpu.VMEM((b_per_w,), jnp.int32),
            pltpu.VMEM((b_per_w, D), jnp.float32),
            pltpu.VMEM((b_per_w, D), jnp.float32),
            pltpu.SemaphoreType.DMA,
            pltpu.SemaphoreType.DMA,
        ],
    )
    def sc_gather(yi_hbm, xi_hbm, yt_hbm, xt_hbm, yout_hbm, xout_hbm,
                  yi_v, xi_v, yrows_v, xrows_v, semy, semx):
        wid = lax.axis_index("s") * NC + lax.axis_index("c")
        base = wid * b_per_w
        pltpu.sync_copy(yi_hbm.at[pl.ds(base, b_per_w)], yi_v)
        pltpu.sync_copy(xi_hbm.at[pl.ds(base, b_per_w)], xi_v)
        cy = pltpu.async_copy(yt_hbm.at[yi_v], yrows_v, semy)
        cx = pltpu.async_copy(xt_hbm.at[xi_v], xrows_v, semx)
        cy.wait()
        cx.wait()
        pltpu.sync_copy(yrows_v, yout_hbm.at[pl.ds(base, b_per_w)])
        pltpu.sync_copy(xrows_v, xout_hbm.at[pl.ds(base, b_per_w)])

    return sc_gather


def _outer_sum_body(S, y_ref, x_ref, o_ref, xt_scr):
    i = pl.program_id(0)

    @pl.when(i == 0)
    def _prologue():
        for j in range(S):
            xt_scr[j] = jnp.swapaxes(x_ref[j], 0, 1)

    yt = jnp.swapaxes(y_ref[0], 0, 1)
    for j in range(S):
        o_ref[0, j] = yt + xt_scr[j]


@functools.lru_cache(maxsize=None)
def _make_outer_sum(N, S, D):
    return pl.pallas_call(
        functools.partial(_outer_sum_body, S),
        grid=(S,),
        in_specs=[
            pl.BlockSpec((1, N, D), lambda i: (i, 0, 0)),
            pl.BlockSpec((S, N, D), lambda i: (0, 0, 0)),
        ],
        out_specs=pl.BlockSpec((1, S, D, N), lambda i: (i, 0, 0, 0)),
        out_shape=jax.ShapeDtypeStruct((S, S, D, N), jnp.float32),
        scratch_shapes=[pltpu.VMEM((S, D, N), jnp.float32)],
    )


def kernel(y_indexes, x_indexes, x_table, y_table):
    N, S = x_indexes.shape
    D = x_table.shape[1]
    B = N * S

    yi = y_indexes.T.reshape(B).astype(jnp.int32)
    xi = x_indexes.T.reshape(B).astype(jnp.int32)

    y2, x2 = _make_sc_gather(B, D)(yi, xi, y_table, x_table)

    out_phys = _make_outer_sum(N, S, D)(
        y2.reshape(S, N, D), x2.reshape(S, N, D))
    return jnp.transpose(out_phys, (3, 2, 0, 1))


# speedup vs baseline: 3.0514x; 1.1239x over previous
import functools

import jax
import jax.numpy as jnp
from jax import lax
from jax.experimental import pallas as pl
from jax.experimental.pallas import tpu as pltpu
from jax.experimental.pallas import tpu_sc as plsc

_LANES = 128


@functools.lru_cache(maxsize=None)
def _make_sc_gather(B):
    info = plsc.get_sparse_core_info()
    NC, NS = info.num_cores, info.num_subcores
    NW = NC * NS
    assert B % (8 * NW) == 0
    b_per_w = B // NW
    mesh = plsc.VectorSubcoreMesh(core_axis_name="c", subcore_axis_name="s")

    @functools.partial(
        pl.kernel,
        mesh=mesh,
        compiler_params=pltpu.CompilerParams(use_tc_tiling_on_sc=False),
        out_type=jax.ShapeDtypeStruct((B, _LANES), jnp.float32),
        scratch_types=[
            pltpu.VMEM((b_per_w,), jnp.int32),
            pltpu.VMEM((b_per_w, _LANES), jnp.float32),
            pltpu.SemaphoreType.DMA,
        ],
    )
    def sc_gather(idx_hbm, tab_hbm, out_hbm, idx_v, rows_v, sem):
        wid = lax.axis_index("s") * NC + lax.axis_index("c")
        base = wid * b_per_w
        pltpu.sync_copy(idx_hbm.at[pl.ds(base, b_per_w)], idx_v)
        pltpu.async_copy(tab_hbm.at[idx_v], rows_v, sem).wait()
        pltpu.sync_copy(rows_v, out_hbm.at[pl.ds(base, b_per_w)])

    return sc_gather


def _outer_sum_body(S, D, y_ref, x_ref, o_ref, xt_scr):
    i = pl.program_id(0)

    @pl.when(i == 0)
    def _prologue():
        for j in range(S):
            xt_scr[j] = jnp.swapaxes(x_ref[j][:, :D], 0, 1)

    yt = jnp.swapaxes(y_ref[0][:, :D], 0, 1)
    for j in range(S):
        o_ref[0, j] = yt + xt_scr[j]


@functools.lru_cache(maxsize=None)
def _make_outer_sum(N, S, D):
    return pl.pallas_call(
        functools.partial(_outer_sum_body, S, D),
        grid=(S,),
        in_specs=[
            pl.BlockSpec((1, N, _LANES), lambda i: (i, 0, 0)),
            pl.BlockSpec((S, N, _LANES), lambda i: (0, 0, 0)),
        ],
        out_specs=pl.BlockSpec((1, S, D, N), lambda i: (i, 0, 0, 0)),
        out_shape=jax.ShapeDtypeStruct((S, S, D, N), jnp.float32),
        scratch_shapes=[pltpu.VMEM((S, D, N), jnp.float32)],
    )


def kernel(y_indexes, x_indexes, x_table, y_table):
    N, S = x_indexes.shape
    D = x_table.shape[1]
    B = N * S

    yi = y_indexes.T.reshape(B).astype(jnp.int32)
    xi = x_indexes.T.reshape(B).astype(jnp.int32)

    yt_w = jnp.pad(y_table, ((0, 0), (0, _LANES - D)))
    xt_w = jnp.pad(x_table, ((0, 0), (0, _LANES - D)))

    gather = _make_sc_gather(B)
    y2 = gather(yi, yt_w)
    x2 = gather(xi, xt_w)

    out_phys = _make_outer_sum(N, S, D)(
        y2.reshape(S, N, _LANES), x2.reshape(S, N, _LANES))
    return jnp.transpose(out_phys, (3, 2, 0, 1))
